# baseline (speedup 1.0000x reference)
import functools

import jax
import jax.numpy as jnp
from jax import lax
from jax.experimental import pallas as pl
from jax.experimental.pallas import tpu as pltpu
from jax.experimental.pallas import tpu_sc as plsc

EMBED = 32
B = 1024
NPLANE = 50 * 20
NC = 2
NS = 16
NW = NC * NS
NUNIT = NPLANE * 4
UPW = NUNIT // NW
UFLOATS = 8 * B
PPW = 32


def _body(idx_hbm, tab_hbm, out_hbm, idx_v, tab_v, buf, sem0, sem1):
    wid = lax.axis_index("s") * NC + lax.axis_index("c")
    u0 = wid * UPW
    pstart = (wid * UPW) // 4
    pltpu.sync_copy(idx_hbm.at[pl.ds(pstart * B, PPW * B)], idx_v)
    pltpu.sync_copy(tab_hbm, tab_v)

    iota = lax.iota(jnp.int32, 16)
    iota2 = iota * 2
    iota2p1 = iota2 + 1

    def do_unit(u, obuf):
        q = u // 4
        dbase = (u % 4) * 8 * 128
        lp = q - pstart

        dvecs = [
            jnp.full((16,), dbase + ds_ * 128, jnp.int32) for ds_ in range(8)
        ]

        @plsc.parallel_loop(0, B // 32, unroll=2)
        def _(bc):
            raw = idx_v[pl.ds(lp * B + bc * 32, 32)]
            iv_e, iv_o = plsc.unpack(raw, format=plsc.PackFormat.INTERLEAVED)
            boff = (bc // 4) * 1024 + (bc % 4) * 32
            for ds_ in range(8):
                v_e = plsc.load_gather(tab_v, [dvecs[ds_] + iv_e])
                v_o = plsc.load_gather(tab_v, [dvecs[ds_] + iv_o])
                sbase = jnp.full((16,), boff + ds_ * 128, jnp.int32)
                plsc.store_scatter(obuf, [sbase + iota2], v_e)
                plsc.store_scatter(obuf, [sbase + iota2p1], v_o)

    def fire(u, obuf, sem):
        q = u // 4
        pout = (q % 50) * 20 + q // 50
        pltpu.async_copy(
            obuf, out_hbm.at[pl.ds((pout * 4 + u % 4) * UFLOATS, UFLOATS)], sem
        )

    def drain(sem):
        pltpu.make_async_copy(
            out_hbm.at[pl.ds(0, UFLOATS)], buf.at[0], sem
        ).wait()

    def pair(i, carry):
        u = u0 + 2 * i

        @pl.when(i > 0)
        def _():
            drain(sem0)

        do_unit(u, buf.at[0])
        fire(u, buf.at[0], sem0)

        @pl.when(i > 0)
        def _():
            drain(sem1)

        do_unit(u + 1, buf.at[1])
        fire(u + 1, buf.at[1], sem1)
        return carry

    lax.fori_loop(0, UPW // 2, pair, 0)
    drain(sem0)
    do_unit(u0 + UPW - 1, buf.at[0])
    fire(u0 + UPW - 1, buf.at[0], sem0)
    drain(sem0)
    drain(sem1)


def kernel(inputs, table):
    b, t, c = inputs.shape
    n = b * t * c
    idx = jnp.transpose(inputs, (2, 1, 0)).reshape(n).astype(jnp.int16)
    tab = table.T.reshape(table.shape[0] * table.shape[1])

    mesh = plsc.VectorSubcoreMesh(
        core_axis_name="c", subcore_axis_name="s", num_cores=NC, num_subcores=NS
    )
    run = pl.kernel(
        _body,
        out_type=jax.ShapeDtypeStruct((n * EMBED,), jnp.float32),
        mesh=mesh,
        scratch_types=[
            pltpu.VMEM((PPW * B,), jnp.int16),
            pltpu.VMEM((tab.shape[0],), jnp.float32),
            pltpu.VMEM((2, UFLOATS), jnp.float32),
            pltpu.SemaphoreType.DMA,
            pltpu.SemaphoreType.DMA,
        ],
        compiler_params=pltpu.CompilerParams(
            use_tc_tiling_on_sc=False, needs_layout_passes=False
        ),
    )
    flat = run(idx, tab)
    out6 = flat.reshape(t, c, EMBED // 8, B // 128, 8, 128)
    return out6.transpose(3, 5, 0, 1, 2, 4).reshape(b, t, c, EMBED)


# speedup vs baseline: 1.0444x; 1.0444x over previous
import functools

import jax
import jax.numpy as jnp
from jax import lax
from jax.experimental import pallas as pl
from jax.experimental.pallas import tpu as pltpu
from jax.experimental.pallas import tpu_sc as plsc

EMBED = 32
B = 1024
NPLANE = 50 * 20
NC = 2
NS = 16
NW = NC * NS
NUNIT = NPLANE * 4
UPW = NUNIT // NW
UFLOATS = 8 * B
PPW = 32


def _body(idx_hbm, tab_hbm, out_hbm, idx_v, tab_v, buf, sem0, sem1):
    wid = lax.axis_index("s") * NC + lax.axis_index("c")
    u0 = wid * UPW
    pstart = (wid * UPW) // 4
    pltpu.sync_copy(idx_hbm.at[pl.ds(pstart * B, PPW * B)], idx_v)
    pltpu.sync_copy(tab_hbm, tab_v)

    iota = lax.iota(jnp.int32, 16)

    def do_unit(u, obuf):
        q = u // 4
        dbase = (u % 4) * 8 * 128
        lp = q - pstart

        dvecs = [
            jnp.full((16,), dbase + ds_ * 128, jnp.int32) for ds_ in range(8)
        ]

        @plsc.parallel_loop(0, B // 16, unroll=2)
        def _(bc):
            iv = idx_v[pl.ds(lp * B + bc * 16, 16)]
            boff = (bc // 8) * 1024 + (bc % 8) * 16
            for ds_ in range(8):
                obuf[pl.ds(boff + ds_ * 128, 16)] = plsc.load_gather(
                    tab_v, [dvecs[ds_] + iv]
                )

    def fire(u, obuf, sem):
        q = u // 4
        pout = (q % 50) * 20 + q // 50
        pltpu.async_copy(
            obuf, out_hbm.at[pl.ds((pout * 4 + u % 4) * UFLOATS, UFLOATS)], sem
        )

    def drain(sem):
        pltpu.make_async_copy(
            out_hbm.at[pl.ds(0, UFLOATS)], buf.at[0], sem
        ).wait()

    def pair(i, carry):
        u = u0 + 2 * i

        @pl.when(i > 0)
        def _():
            drain(sem0)

        do_unit(u, buf.at[0])
        fire(u, buf.at[0], sem0)

        @pl.when(i > 0)
        def _():
            drain(sem1)

        do_unit(u + 1, buf.at[1])
        fire(u + 1, buf.at[1], sem1)
        return carry

    lax.fori_loop(0, UPW // 2, pair, 0)
    drain(sem0)
    do_unit(u0 + UPW - 1, buf.at[0])
    fire(u0 + UPW - 1, buf.at[0], sem0)
    drain(sem0)
    drain(sem1)


def kernel(inputs, table):
    b, t, c = inputs.shape
    n = b * t * c
    idx = jnp.transpose(inputs, (2, 1, 0)).reshape(n).astype(jnp.int32)
    tab = table.T.reshape(table.shape[0] * table.shape[1])

    mesh = plsc.VectorSubcoreMesh(
        core_axis_name="c", subcore_axis_name="s", num_cores=NC, num_subcores=NS
    )
    run = pl.kernel(
        _body,
        out_type=jax.ShapeDtypeStruct((n * EMBED,), jnp.float32),
        mesh=mesh,
        scratch_types=[
            pltpu.VMEM((PPW * B,), jnp.int32),
            pltpu.VMEM((tab.shape[0],), jnp.float32),
            pltpu.VMEM((2, UFLOATS), jnp.float32),
            pltpu.SemaphoreType.DMA,
            pltpu.SemaphoreType.DMA,
        ],
        compiler_params=pltpu.CompilerParams(
            use_tc_tiling_on_sc=False, needs_layout_passes=False
        ),
    )
    flat = run(idx, tab)
    out6 = flat.reshape(t, c, EMBED // 8, B // 128, 8, 128)
    return out6.transpose(3, 5, 0, 1, 2, 4).reshape(b, t, c, EMBED)
